# Initial kernel scaffold; baseline (speedup 1.0000x reference)
import jax
import jax.numpy as jnp
from jax.experimental import pallas as pl


def kernel(minhashes):
    raise NotImplementedError("write your pallas kernel here")


# speedup vs baseline: 6.0995x; 6.0995x over previous
import functools

import jax
import jax.numpy as jnp
from jax import lax
from jax.experimental import pallas as pl
from jax.experimental.pallas import tpu as pltpu
from jax.experimental.pallas import tpu_sc as plsc

B = 16
S_LEN = 128
N_HASH = 64
M_BLOOM = 1024
W_WIN = 3
NBLK = 2 * W_WIN + 1

LANES = 16
NUM_CORES = 2
NUM_SUBCORES = 16
MH = M_BLOOM // 2
PAD = 8
PADW = PAD + S_LEN + 8
SBLKS = S_LEN // LANES
GROUP = 32
NGRP = MH // GROUP


def _body(mh_hbm, out_hbm, inp, cnt, stg, sem0, sem1, sem2):
    wid = lax.axis_index("s") * NUM_CORES + lax.axis_index("c")
    b = wid // 2
    m_base = (wid % 2) * MH

    in_copy = pltpu.make_async_copy(mh_hbm.at[b], inp, sem0)
    in_copy.start()

    zeros = jnp.zeros((LANES,), jnp.float32)

    def zrow(r, _):
        for j in range(PADW // LANES):
            cnt[r, pl.ds(j * LANES, LANES)] = zeros
        return 0

    lax.fori_loop(0, MH, zrow, 0)

    in_copy.wait()

    iota = lax.iota(jnp.int32, LANES)
    ones = jnp.ones((LANES,), jnp.float32)

    def scat(i, _):
        n = i // SBLKS
        sb = i - n * SBLKS
        s_vec = sb * LANES + iota
        n_vec = jnp.full((LANES,), n, jnp.int32)
        h = plsc.load_gather(inp, [s_vec, n_vec])
        rel = (h & (M_BLOOM - 1)) - m_base
        mask = (rel >= 0) & (rel < MH)
        rel_safe = jnp.where(mask, rel, 0)
        plsc.addupdate_scatter(cnt, [rel_safe, PAD + s_vec], ones, mask=mask)
        return 0

    lax.fori_loop(0, N_HASH * SBLKS, scat, 0)

    k3_copy = pltpu.make_async_copy(
        cnt.at[:, pl.ds(PAD, S_LEN)],
        out_hbm.at[b, pl.ds(3 * M_BLOOM + m_base, MH), :],
        sem2,
    )
    k3_copy.start()

    sems = (sem0, sem1)

    def _wait_stg(half):
        pltpu.make_async_copy(
            stg.at[half],
            out_hbm.at[b, pl.ds(0, GROUP), :],
            sems[half],
        ).wait()

    kk = 0
    for k in range(NBLK):
        if k == 3:
            continue
        c0 = 5 + k
        first_k = kk == 0

        def emit_pair(i, _, c0=c0, k=k, first_k=first_k):
            for half in range(2):
                g = i * 2 + half
                row0 = g * GROUP

                if first_k:
                    @pl.when(i >= 1)
                    def _w():
                        _wait_stg(half)
                else:
                    _wait_stg(half)

                def load_row(rr):
                    return [
                        cnt[row0 + rr, pl.ds(c0 + j * LANES, LANES)]
                        for j in range(SBLKS)
                    ]

                def store_row(rr, vals):
                    for j in range(SBLKS):
                        stg[half, rr, pl.ds(j * LANES, LANES)] = vals[j]

                prev = load_row(0)
                for rr in range(1, GROUP):
                    cur = load_row(rr)
                    store_row(rr - 1, prev)
                    prev = cur
                store_row(GROUP - 1, prev)

                pltpu.make_async_copy(
                    stg.at[half],
                    out_hbm.at[b, pl.ds(k * M_BLOOM + m_base + row0, GROUP), :],
                    sems[half],
                ).start()
            return 0

        lax.fori_loop(0, NGRP // 2, emit_pair, 0)
        kk += 1

    for half in range(2):
        _wait_stg(half)
    k3_copy.wait()


@functools.partial(jax.jit, static_argnames=())
def kernel(minhashes):
    mesh = plsc.VectorSubcoreMesh(
        core_axis_name="c", subcore_axis_name="s",
        num_cores=NUM_CORES, num_subcores=NUM_SUBCORES,
    )
    run = pl.kernel(
        _body,
        out_type=jax.ShapeDtypeStruct((B, NBLK * M_BLOOM, S_LEN), jnp.float32),
        mesh=mesh,
        scratch_types=[
            pltpu.VMEM((S_LEN, N_HASH), jnp.int32),
            pltpu.VMEM((MH, PADW), jnp.float32),
            pltpu.VMEM((2, GROUP, S_LEN), jnp.float32),
            pltpu.SemaphoreType.DMA,
            pltpu.SemaphoreType.DMA,
            pltpu.SemaphoreType.DMA,
        ],
        compiler_params=pltpu.CompilerParams(
            use_tc_tiling_on_sc=False, needs_layout_passes=False
        ),
    )
    return run(minhashes)


# speedup vs baseline: 7.2824x; 1.1939x over previous
import functools

import jax
import jax.numpy as jnp
from jax import lax
from jax.experimental import pallas as pl
from jax.experimental.pallas import tpu as pltpu
from jax.experimental.pallas import tpu_sc as plsc

B = 16
S_LEN = 128
N_HASH = 64
M_BLOOM = 1024
W_WIN = 3
NBLK = 2 * W_WIN + 1

LANES = 16
NUM_CORES = 2
NUM_SUBCORES = 16
MH = M_BLOOM // 2
SBLKS = S_LEN // LANES


def _hist_body(mh_hbm, cnt_hbm, inp, cnt, sem):
    wid = lax.axis_index("s") * NUM_CORES + lax.axis_index("c")
    b = wid // 2
    m_base = (wid % 2) * MH

    in_copy = pltpu.make_async_copy(mh_hbm.at[b], inp, sem)
    in_copy.start()

    zeros = jnp.zeros((LANES,), jnp.float32)

    def zrow(r, _):
        for j in range(S_LEN // LANES):
            cnt[r, pl.ds(j * LANES, LANES)] = zeros
        return 0

    lax.fori_loop(0, MH, zrow, 0)
    in_copy.wait()

    iota = lax.iota(jnp.int32, LANES)
    ones = jnp.ones((LANES,), jnp.float32)

    def scat(i, _):
        n = i // SBLKS
        sb = i - n * SBLKS
        s_vec = sb * LANES + iota
        n_vec = jnp.full((LANES,), n, jnp.int32)
        h = plsc.load_gather(inp, [s_vec, n_vec])
        rel = (h & (M_BLOOM - 1)) - m_base
        mask = (rel >= 0) & (rel < MH)
        rel_safe = jnp.where(mask, rel, 0)
        plsc.addupdate_scatter(cnt, [rel_safe, s_vec], ones, mask=mask)
        return 0

    lax.fori_loop(0, N_HASH * SBLKS, scat, 0)

    pltpu.sync_copy(cnt, cnt_hbm.at[b, pl.ds(m_base, MH), :])


def _sc_histogram(minhashes):
    mesh = plsc.VectorSubcoreMesh(
        core_axis_name="c", subcore_axis_name="s",
        num_cores=NUM_CORES, num_subcores=NUM_SUBCORES,
    )
    run = pl.kernel(
        _hist_body,
        out_type=jax.ShapeDtypeStruct((B, M_BLOOM, S_LEN), jnp.float32),
        mesh=mesh,
        scratch_types=[
            pltpu.VMEM((S_LEN, N_HASH), jnp.int32),
            pltpu.VMEM((MH, S_LEN), jnp.float32),
            pltpu.SemaphoreType.DMA,
        ],
        compiler_params=pltpu.CompilerParams(
            use_tc_tiling_on_sc=False, needs_layout_passes=False
        ),
    )
    return run(minhashes)


def _emit_body(cin, cout):
    x = cin[0]
    for k in range(NBLK):
        d = W_WIN - k
        if d > 0:
            blk = jnp.concatenate(
                [jnp.zeros((M_BLOOM, d), jnp.float32), x[:, : S_LEN - d]], axis=1
            )
        elif d == 0:
            blk = x
        else:
            e = -d
            blk = jnp.concatenate(
                [x[:, e:], jnp.zeros((M_BLOOM, e), jnp.float32)], axis=1
            )
        cout[0, k * M_BLOOM : (k + 1) * M_BLOOM, :] = blk


def _tc_emit(counts):
    return pl.pallas_call(
        _emit_body,
        out_shape=jax.ShapeDtypeStruct((B, NBLK * M_BLOOM, S_LEN), jnp.float32),
        grid=(B,),
        in_specs=[pl.BlockSpec((1, M_BLOOM, S_LEN), lambda i: (i, 0, 0))],
        out_specs=pl.BlockSpec((1, NBLK * M_BLOOM, S_LEN), lambda i: (i, 0, 0)),
    )(counts)


@functools.partial(jax.jit, static_argnames=())
def kernel(minhashes):
    return _tc_emit(_sc_histogram(minhashes))
